# Initial kernel scaffold; baseline (speedup 1.0000x reference)
import functools

import jax
import jax.numpy as jnp
from jax import lax
from jax.experimental import pallas as pl
from jax.experimental.pallas import tpu as pltpu
from jax.experimental.pallas import tpu_sc as plsc

N = 10000
E = 160000
IN = 256
H = 4
OUT = 64
HO = H * OUT
NEG = 0.2

NP = 10240
NW = 32
W = 128
CHUNK = 5376
NWIN = CHUNK // W
EP = NW * CHUNK
RPT = NP // 16


def _dense_body(x_ref, w_ref, a_ref, xw0_ref, xw1_ref, s_ref):
    xw = jnp.dot(x_ref[...], w_ref[...], preferred_element_type=jnp.float32)
    xw0_ref[...] = xw[:, :128]
    xw1_ref[...] = xw[:, 128:]
    s_ref[...] = jnp.dot(xw, a_ref[...], preferred_element_type=jnp.float32)


def _dense(xpad, weight, amat):
    bm = 512
    grid = NP // bm
    return pl.pallas_call(
        _dense_body,
        grid=(grid,),
        in_specs=[
            pl.BlockSpec((bm, IN), lambda i: (i, 0)),
            pl.BlockSpec((IN, HO), lambda i: (0, 0)),
            pl.BlockSpec((IN, 8), lambda i: (0, 0)),
        ],
        out_specs=[
            pl.BlockSpec((bm, 128), lambda i: (i, 0)),
            pl.BlockSpec((bm, 128), lambda i: (i, 0)),
            pl.BlockSpec((bm, 8), lambda i: (i, 0)),
        ],
        out_shape=[
            jax.ShapeDtypeStruct((NP, 128), jnp.float32),
            jax.ShapeDtypeStruct((NP, 128), jnp.float32),
            jax.ShapeDtypeStruct((NP, 8), jnp.float32),
        ],
    )(xpad, weight, amat)


def _sc_body(src_hbm, dst_hbm, stab_hbm, xw0_hbm, xw1_hbm,
             num_out, den_out,
             stab_v, srcwin, dstwin, rowbuf, pbuf, dbuf, denstage,
             acc_s, den_s, sem):
    c = lax.axis_index("c")
    s = lax.axis_index("s")
    base = (c * 16 + s) * CHUNK
    zeros16 = jnp.zeros((16,), jnp.float32)

    pltpu.sync_copy(stab_hbm, stab_v)

    def zero_rowbuf():
        def zb(i, carry):
            for k in range(8):
                rowbuf[i, pl.ds(k * 16, 16)] = zeros16
            return carry
        lax.fori_loop(0, W, zb, 0)

    def zero_acc_rows():
        for r in range(RPT // W):
            pltpu.sync_copy(rowbuf, acc_s.at[pl.ds(s * RPT + r * W, W)])

    zero_rowbuf()
    zero_acc_rows()

    def zd(i, carry):
        dbuf[i, :] = zeros16
        return carry
    lax.fori_loop(0, W, zd, 0)

    def zd2(i, carry):
        denstage[i, :] = zeros16
        return carry
    lax.fori_loop(0, RPT, zd2, 0)
    pltpu.sync_copy(denstage, den_s.at[pl.ds(s * RPT, RPT)])
    plsc.subcore_barrier()

    for p in (0, 1):
        xw_hbm = xw0_hbm if p == 0 else xw1_hbm

        def window(wi, carry):
            off = base + wi * W
            pltpu.sync_copy(src_hbm.at[pl.ds(off, W)], srcwin)
            pltpu.sync_copy(dst_hbm.at[pl.ds(off, W)], dstwin)
            pltpu.async_copy(xw_hbm.at[srcwin], rowbuf, sem).wait()

            for g in range(W // 16):
                sv = srcwin[pl.ds(g * 16, 16)]
                dv = dstwin[pl.ds(g * 16, 16)]
                ev = lax.iota(jnp.int32, 16) + g * 16
                sb = sv * 8
                db = dv * 8 + 4
                for h in (range(4) if p == 0 else (2, 3)):
                    sj = plsc.load_gather(stab_v, [sb + h])
                    si = plsc.load_gather(stab_v, [db + h])
                    a = si + sj
                    a = jnp.where(a >= 0.0, a, NEG * a)
                    ph = jnp.exp(a)
                    if p == 0:
                        plsc.store_scatter(
                            dbuf, [ev, jnp.full((16,), h, jnp.int32)], ph)
                    hl = h - 2 * p
                    if 0 <= hl < 2:
                        plsc.store_scatter(
                            pbuf, [jnp.full((16,), hl, jnp.int32), ev], ph)

            def scale(e, carry2):
                p0 = jnp.broadcast_to(pbuf[0, e], (16,))
                p1 = jnp.broadcast_to(pbuf[1, e], (16,))
                for k in range(4):
                    rowbuf[e, pl.ds(k * 16, 16)] = (
                        rowbuf[e, pl.ds(k * 16, 16)] * p0)
                for k in range(4):
                    rowbuf[e, pl.ds(64 + k * 16, 16)] = (
                        rowbuf[e, pl.ds(64 + k * 16, 16)] * p1)
                return carry2
            lax.fori_loop(0, W, scale, 0)

            pltpu.sync_copy(rowbuf, acc_s.at[dstwin], add=True)
            if p == 0:
                pltpu.sync_copy(dbuf, den_s.at[dstwin], add=True)
            return carry

        lax.fori_loop(0, NWIN, window, 0)
        plsc.subcore_barrier()

        for r in range(RPT // W):
            rows = pl.ds(s * RPT + r * W, W)
            pltpu.sync_copy(acc_s.at[rows], rowbuf)
            pltpu.sync_copy(rowbuf, num_out.at[p, c, rows])

        if p == 0:
            zero_rowbuf()
            zero_acc_rows()
            plsc.subcore_barrier()

    pltpu.sync_copy(den_s.at[pl.ds(s * RPT, RPT)], denstage)
    pltpu.sync_copy(denstage, den_out.at[c, pl.ds(s * RPT, RPT)])


_sc_edges = functools.partial(
    pl.kernel,
    _sc_body,
    mesh=plsc.VectorSubcoreMesh(core_axis_name="c", subcore_axis_name="s"),
    out_type=[
        jax.ShapeDtypeStruct((2, 2, NP, 128), jnp.float32),
        jax.ShapeDtypeStruct((2, NP, 16), jnp.float32),
    ],
    scratch_types=[
        pltpu.VMEM((NP * 8,), jnp.float32),
        pltpu.VMEM((W,), jnp.int32),
        pltpu.VMEM((W,), jnp.int32),
        pltpu.VMEM((W, 128), jnp.float32),
        pltpu.VMEM((2, W), jnp.float32),
        pltpu.VMEM((W, 16), jnp.float32),
        pltpu.VMEM((RPT, 16), jnp.float32),
        pltpu.VMEM_SHARED((NP, 128), jnp.float32),
        pltpu.VMEM_SHARED((NP, 16), jnp.float32),
        pltpu.SemaphoreType.DMA,
    ],
)


def _finish_body(num_ref, den_ref, bias_ref, out_ref):
    bm = out_ref.shape[0]
    n0 = num_ref[0] + num_ref[1]
    n1 = num_ref[2] + num_ref[3]
    den = den_ref[0] + den_ref[1]
    d0 = jnp.concatenate(
        [jnp.broadcast_to(den[:, h:h + 1], (bm, OUT)) for h in (0, 1)], axis=1)
    d1 = jnp.concatenate(
        [jnp.broadcast_to(den[:, h:h + 1], (bm, OUT)) for h in (2, 3)], axis=1)
    out_ref[...] = jnp.concatenate(
        [n0 / (d0 + 1e-16), n1 / (d1 + 1e-16)], axis=1) + bias_ref[...]


def _finish(num, den, bias2d):
    bm = 400
    grid = N // bm
    return pl.pallas_call(
        _finish_body,
        grid=(grid,),
        in_specs=[
            pl.BlockSpec((4, bm, 128), lambda i: (0, i, 0)),
            pl.BlockSpec((2, bm, 16), lambda i: (0, i, 0)),
            pl.BlockSpec((1, HO), lambda i: (0, 0)),
        ],
        out_specs=pl.BlockSpec((bm, HO), lambda i: (i, 0)),
        out_shape=jax.ShapeDtypeStruct((N, HO), jnp.float32),
    )(num, den, bias2d)


def kernel(x, edge_index, weight, att, bias):
    src0, dst0 = edge_index[0], edge_index[1]
    keep = src0 != dst0
    npv = jnp.asarray(N, src0.dtype)
    src0 = jnp.where(keep, src0, npv)
    dst0 = jnp.where(keep, dst0, npv)
    loop = jnp.arange(N, dtype=src0.dtype)
    npad = EP - (E + N)
    padr = (N + 1 + jnp.arange(npad, dtype=src0.dtype) % (NP - N - 1))
    src = jnp.concatenate([src0, loop, padr])
    dst = jnp.concatenate([dst0, loop, padr])

    att_i = att[0, :, :OUT]
    att_j = att[0, :, OUT:]
    eye = jnp.eye(H, dtype=jnp.float32)
    a_j = (att_j[:, :, None] * eye[:, None, :]).reshape(HO, H)
    a_i = (att_i[:, :, None] * eye[:, None, :]).reshape(HO, H)
    amat = jnp.concatenate([a_j, a_i], axis=1)

    xpad = jnp.pad(x, ((0, NP - N), (0, 0)))

    xw0, xw1, stab = _dense(xpad, weight, amat)
    stab_flat = stab.reshape(NP * 8)

    num, den = _sc_edges(src.astype(jnp.int32), dst.astype(jnp.int32),
                         stab_flat, xw0, xw1)

    out = _finish(num.reshape(4, NP, 128), den, bias.reshape(1, HO))

    ixz = jnp.zeros((N,), dtype=x.dtype)
    structure_kl_loss = jnp.zeros((), dtype=x.dtype)
    return (out, ixz, structure_kl_loss)


# speedup vs baseline: 43.8078x; 43.8078x over previous
import functools

import jax
import jax.numpy as jnp
from jax import lax
from jax.experimental import pallas as pl
from jax.experimental.pallas import tpu as pltpu
from jax.experimental.pallas import tpu_sc as plsc

N = 10000
E = 160000
IN = 256
H = 4
OUT = 64
HO = H * OUT
NEG = 0.2

NP = 10240
NW = 32
W = 128
CHUNK = 5376
NWIN = CHUNK // W
EP = NW * CHUNK
RPT = NP // 16


def _dense_body(x_ref, w_ref, a_ref, xw0_ref, xw1_ref, s_ref):
    xw = jnp.dot(x_ref[...], w_ref[...], preferred_element_type=jnp.float32)
    xw0_ref[...] = xw[:, :128]
    xw1_ref[...] = xw[:, 128:]
    s_ref[...] = jnp.dot(xw, a_ref[...], preferred_element_type=jnp.float32)


def _dense(xpad, weight, amat):
    bm = 512
    grid = NP // bm
    return pl.pallas_call(
        _dense_body,
        grid=(grid,),
        in_specs=[
            pl.BlockSpec((bm, IN), lambda i: (i, 0)),
            pl.BlockSpec((IN, HO), lambda i: (0, 0)),
            pl.BlockSpec((IN, 16), lambda i: (0, 0)),
        ],
        out_specs=[
            pl.BlockSpec((bm, 128), lambda i: (i, 0)),
            pl.BlockSpec((bm, 128), lambda i: (i, 0)),
            pl.BlockSpec((bm, 16), lambda i: (i, 0)),
        ],
        out_shape=[
            jax.ShapeDtypeStruct((NP, 128), jnp.float32),
            jax.ShapeDtypeStruct((NP, 128), jnp.float32),
            jax.ShapeDtypeStruct((NP, 16), jnp.float32),
        ],
    )(xpad, weight, amat)


def _sc_body(src_hbm, dst_hbm, stab_hbm, xw0_hbm, xw1_hbm,
             num_out, den_out,
             srcwin, dstwin, rowbuf, pbuf, dbuf, sjb, sib,
             acc_s, den_s, sem):
    c = lax.axis_index("c")
    s = lax.axis_index("s")
    base = (c * 16 + s) * CHUNK
    zeros16 = jnp.zeros((16,), jnp.float32)

    def zero_rowbuf():
        def zb(i, carry):
            for k in range(8):
                rowbuf[i, pl.ds(k * 16, 16)] = zeros16
            return carry
        lax.fori_loop(0, W, zb, 0)

    def zero_acc_rows():
        for r in range(RPT // W):
            pltpu.sync_copy(rowbuf, acc_s.at[pl.ds(s * RPT + r * W, W)])

    zero_rowbuf()
    zero_acc_rows()

    def zd(i, carry):
        dbuf[i, :] = zeros16
        return carry
    lax.fori_loop(0, W, zd, 0)
    for r in range(RPT // W):
        pltpu.sync_copy(dbuf, den_s.at[pl.ds(s * RPT + r * W, W)])
    plsc.subcore_barrier()

    for p in (0, 1):
        xw_hbm = xw0_hbm if p == 0 else xw1_hbm

        def window(wi, carry):
            off = base + wi * W
            pltpu.sync_copy(src_hbm.at[pl.ds(off, W)], srcwin)
            pltpu.sync_copy(dst_hbm.at[pl.ds(off, W)], dstwin)
            h1 = pltpu.async_copy(xw_hbm.at[srcwin], rowbuf, sem)
            h2 = pltpu.async_copy(stab_hbm.at[srcwin], sjb, sem)
            h3 = pltpu.async_copy(stab_hbm.at[dstwin], sib, sem)
            h1.wait()
            h2.wait()
            h3.wait()

            for g in range(W // 16):
                ev = lax.iota(jnp.int32, 16) + g * 16
                for h in (range(4) if p == 0 else (2, 3)):
                    hv = jnp.full((16,), h, jnp.int32)
                    sj = plsc.load_gather(sjb, [ev, hv])
                    si = plsc.load_gather(sib, [ev, hv + 4])
                    a = si + sj
                    a = jnp.where(a >= 0.0, a, NEG * a)
                    ph = jnp.exp(a)
                    if p == 0:
                        plsc.store_scatter(
                            dbuf, [ev, jnp.full((16,), h, jnp.int32)], ph)
                    hl = h - 2 * p
                    if 0 <= hl < 2:
                        plsc.store_scatter(pbuf, [ev + hl * W], ph)

            def scale(e, carry2):
                e16 = jnp.broadcast_to(e, (16,)).astype(jnp.int32)
                p0 = plsc.load_gather(pbuf, [e16])
                p1 = plsc.load_gather(pbuf, [e16 + W])
                for k in range(4):
                    rowbuf[e, pl.ds(k * 16, 16)] = (
                        rowbuf[e, pl.ds(k * 16, 16)] * p0)
                for k in range(4):
                    rowbuf[e, pl.ds(64 + k * 16, 16)] = (
                        rowbuf[e, pl.ds(64 + k * 16, 16)] * p1)
                return carry2
            lax.fori_loop(0, W, scale, 0)

            pltpu.sync_copy(rowbuf, acc_s.at[dstwin], add=True)
            if p == 0:
                pltpu.sync_copy(dbuf, den_s.at[dstwin], add=True)
            return carry

        lax.fori_loop(0, NWIN, window, 0)
        plsc.subcore_barrier()

        for r in range(RPT // W):
            rows = pl.ds(s * RPT + r * W, W)
            pltpu.sync_copy(acc_s.at[rows], rowbuf)
            pltpu.sync_copy(rowbuf, num_out.at[p, c, rows])

        if p == 0:
            zero_rowbuf()
            zero_acc_rows()
            plsc.subcore_barrier()

    for r in range(RPT // W):
        rows = pl.ds(s * RPT + r * W, W)
        pltpu.sync_copy(den_s.at[rows], dbuf)
        pltpu.sync_copy(dbuf, den_out.at[c, rows])


@functools.cache
def _make_sc_edges():
    return pl.kernel(
        _sc_body,
        mesh=plsc.VectorSubcoreMesh(core_axis_name="c", subcore_axis_name="s"),
        compiler_params=pltpu.CompilerParams(
            needs_layout_passes=False, use_tc_tiling_on_sc=False),
        out_type=[
            jax.ShapeDtypeStruct((2, 2, NP, 128), jnp.float32),
            jax.ShapeDtypeStruct((2, NP, 16), jnp.float32),
        ],
        scratch_types=[
            pltpu.VMEM((W,), jnp.int32),
            pltpu.VMEM((W,), jnp.int32),
            pltpu.VMEM((W, 128), jnp.float32),
            pltpu.VMEM((2 * W,), jnp.float32),
            pltpu.VMEM((W, 16), jnp.float32),
            pltpu.VMEM((W, 16), jnp.float32),
            pltpu.VMEM((W, 16), jnp.float32),
            pltpu.VMEM_SHARED((NP, 128), jnp.float32),
            pltpu.VMEM_SHARED((NP, 16), jnp.float32),
            pltpu.SemaphoreType.DMA,
        ],
    )


def _finish_body(num_ref, den_ref, bias_ref, out_ref):
    bm = out_ref.shape[0]
    n0 = num_ref[0] + num_ref[1]
    n1 = num_ref[2] + num_ref[3]
    den = den_ref[0] + den_ref[1]
    d0 = jnp.concatenate(
        [jnp.broadcast_to(den[:, h:h + 1], (bm, OUT)) for h in (0, 1)], axis=1)
    d1 = jnp.concatenate(
        [jnp.broadcast_to(den[:, h:h + 1], (bm, OUT)) for h in (2, 3)], axis=1)
    out_ref[...] = jnp.concatenate(
        [n0 / (d0 + 1e-16), n1 / (d1 + 1e-16)], axis=1) + bias_ref[...]


def _finish(num, den, bias2d):
    bm = 400
    grid = N // bm
    return pl.pallas_call(
        _finish_body,
        grid=(grid,),
        in_specs=[
            pl.BlockSpec((4, bm, 128), lambda i: (0, i, 0)),
            pl.BlockSpec((2, bm, 16), lambda i: (0, i, 0)),
            pl.BlockSpec((1, HO), lambda i: (0, 0)),
        ],
        out_specs=pl.BlockSpec((bm, HO), lambda i: (i, 0)),
        out_shape=jax.ShapeDtypeStruct((N, HO), jnp.float32),
    )(num, den, bias2d)


def kernel(x, edge_index, weight, att, bias):
    src0, dst0 = edge_index[0], edge_index[1]
    keep = src0 != dst0
    npv = jnp.asarray(N, src0.dtype)
    src0 = jnp.where(keep, src0, npv)
    dst0 = jnp.where(keep, dst0, npv)
    loop = jnp.arange(N, dtype=src0.dtype)
    npad = EP - (E + N)
    padr = (N + 1 + jnp.arange(npad, dtype=src0.dtype) % (NP - N - 1))
    src = jnp.concatenate([src0, loop, padr])
    dst = jnp.concatenate([dst0, loop, padr])

    att_i = att[0, :, :OUT]
    att_j = att[0, :, OUT:]
    eye = jnp.eye(H, dtype=jnp.float32)
    a_j = (att_j[:, :, None] * eye[:, None, :]).reshape(HO, H)
    a_i = (att_i[:, :, None] * eye[:, None, :]).reshape(HO, H)
    amat = jnp.concatenate(
        [a_j, a_i, jnp.zeros((HO, 8), jnp.float32)], axis=1)

    xpad = jnp.pad(x, ((0, NP - N), (0, 0)))

    xw0, xw1, stab = _dense(xpad, weight, amat)

    num, den = _make_sc_edges()(src.astype(jnp.int32), dst.astype(jnp.int32),
                                stab, xw0, xw1)

    out = _finish(num.reshape(4, NP, 128), den, bias.reshape(1, HO))

    ixz = jnp.zeros((N,), dtype=x.dtype)
    structure_kl_loss = jnp.zeros((), dtype=x.dtype)
    return (out, ixz, structure_kl_loss)


# speedup vs baseline: 62.6993x; 1.4312x over previous
import functools

import jax
import jax.numpy as jnp
from jax import lax
from jax.experimental import pallas as pl
from jax.experimental.pallas import tpu as pltpu
from jax.experimental.pallas import tpu_sc as plsc

N = 10000
E = 160000
IN = 256
H = 4
OUT = 64
HO = H * OUT
NEG = 0.2

NP = 10240
NW = 32
W = 64
CHUNK = 5376
NWIN = CHUNK // W
NPAIR = NWIN // 2
EP = NW * CHUNK
RPT = NP // 16


def _dense_body(x_ref, w_ref, a_ref, xw0_ref, xw1_ref, s_ref):
    xw = jnp.dot(x_ref[...], w_ref[...], preferred_element_type=jnp.float32)
    xw0_ref[...] = xw[:, :128]
    xw1_ref[...] = xw[:, 128:]
    s_ref[...] = jnp.dot(xw, a_ref[...], preferred_element_type=jnp.float32)


def _dense(xpad, weight, amat):
    bm = 512
    grid = NP // bm
    return pl.pallas_call(
        _dense_body,
        grid=(grid,),
        in_specs=[
            pl.BlockSpec((bm, IN), lambda i: (i, 0)),
            pl.BlockSpec((IN, HO), lambda i: (0, 0)),
            pl.BlockSpec((IN, 16), lambda i: (0, 0)),
        ],
        out_specs=[
            pl.BlockSpec((bm, 128), lambda i: (i, 0)),
            pl.BlockSpec((bm, 128), lambda i: (i, 0)),
            pl.BlockSpec((bm, 16), lambda i: (i, 0)),
        ],
        out_shape=[
            jax.ShapeDtypeStruct((NP, 128), jnp.float32),
            jax.ShapeDtypeStruct((NP, 128), jnp.float32),
            jax.ShapeDtypeStruct((NP, 16), jnp.float32),
        ],
    )(xpad, weight, amat)


def _sc_body(src_hbm, dst_hbm, stab_hbm, xw0_hbm, xw1_hbm,
             num_out, den_out,
             srcall, dstall,
             rowbuf_a, rowbuf_b, sjb_a, sjb_b, sib_a, sib_b,
             dstwin_a, dstwin_b, dbuf_a, dbuf_b, pbuf,
             acc_s, den_s, sem_ga, sem_gb, sem_sa, sem_sb):
    c = lax.axis_index("c")
    s = lax.axis_index("s")
    wid = c * 16 + s
    zeros16 = jnp.zeros((16,), jnp.float32)

    slot_a = (rowbuf_a, sjb_a, sib_a, dstwin_a, dbuf_a, sem_ga, sem_sa)
    slot_b = (rowbuf_b, sjb_b, sib_b, dstwin_b, dbuf_b, sem_gb, sem_sb)

    pltpu.sync_copy(src_hbm.at[wid], srcall)
    pltpu.sync_copy(dst_hbm.at[wid], dstall)

    def zero_rowbuf(rb):
        def zb(i, carry):
            for k in range(8):
                rb[i, pl.ds(k * 16, 16)] = zeros16
            return carry
        lax.fori_loop(0, W, zb, 0)

    def zero_acc_rows():
        for r in range(RPT // W):
            pltpu.sync_copy(rowbuf_a, acc_s.at[pl.ds(s * RPT + r * W, W)])

    zero_rowbuf(rowbuf_a)
    zero_acc_rows()

    def zd(i, carry):
        dbuf_a[i, :] = zeros16
        dbuf_b[i, :] = zeros16
        return carry
    lax.fori_loop(0, W, zd, 0)
    for r in range(RPT // W):
        pltpu.sync_copy(dbuf_a, den_s.at[pl.ds(s * RPT + r * W, W)])
    plsc.subcore_barrier()

    for p in (0, 1):
        xw_hbm = xw0_hbm if p == 0 else xw1_hbm

        def fire_gather(kw, sl):
            rb, sj, si, _, _, sg, _ = sl
            pltpu.async_copy(xw_hbm.at[srcall.at[kw, 0]], rb, sg)
            pltpu.async_copy(stab_hbm.at[srcall.at[kw, 0]], sj, sg)
            pltpu.async_copy(stab_hbm.at[dstall.at[kw, 0]], si, sg)

        def wait_gather(kw, sl):
            rb, sj, si, _, _, sg, _ = sl
            pltpu.make_async_copy(xw_hbm.at[srcall.at[kw, 0]], rb, sg).wait()
            pltpu.make_async_copy(stab_hbm.at[srcall.at[kw, 0]], sj, sg).wait()
            pltpu.make_async_copy(stab_hbm.at[dstall.at[kw, 0]], si, sg).wait()

        def fire_scatter(sl):
            rb, _, _, dw, db, _, ss = sl
            pltpu.async_copy(rb, acc_s.at[dw], ss, add=True)
            if p == 0:
                pltpu.async_copy(db, den_s.at[dw], ss, add=True)

        def drain_scatter(sl):
            rb, _, _, dw, db, _, ss = sl
            pltpu.make_async_copy(rb, acc_s.at[dw], ss).wait()
            if p == 0:
                pltpu.make_async_copy(db, den_s.at[dw], ss).wait()

        def compute(kw, sl):
            rb, sj_b, si_b, dw, db, _, _ = sl
            for g in range(W // 16):
                dw[pl.ds(g * 16, 16)] = dstall[kw, 0, pl.ds(g * 16, 16)]
            for g in range(W // 16):
                ev = lax.iota(jnp.int32, 16) + g * 16
                for h in (range(4) if p == 0 else (2, 3)):
                    hv = jnp.full((16,), h, jnp.int32)
                    sj = plsc.load_gather(sj_b, [ev, hv])
                    si = plsc.load_gather(si_b, [ev, hv + 4])
                    a = si + sj
                    a = jnp.where(a >= 0.0, a, NEG * a)
                    ph = jnp.exp(a)
                    if p == 0:
                        plsc.store_scatter(db, [ev, hv], ph)
                    hl = h - 2 * p
                    if 0 <= hl < 2:
                        plsc.store_scatter(pbuf, [ev + hl * W], ph)
            def scale(e, carry2):
                e16 = jnp.broadcast_to(e, (16,)).astype(jnp.int32)
                p0 = plsc.load_gather(pbuf, [e16])
                p1 = plsc.load_gather(pbuf, [e16 + W])
                for k in range(4):
                    rb[e, pl.ds(k * 16, 16)] = rb[e, pl.ds(k * 16, 16)] * p0
                for k in range(4):
                    rb[e, pl.ds(64 + k * 16, 16)] = (
                        rb[e, pl.ds(64 + k * 16, 16)] * p1)
                return carry2
            lax.fori_loop(0, W, scale, 0)

        fire_gather(jnp.int32(0), slot_a)

        def pair(k, carry):
            ka = 2 * k
            kb = ka + 1
            fire_gather(kb, slot_b)
            wait_gather(ka, slot_a)
            compute(ka, slot_a)
            fire_scatter(slot_a)
            wait_gather(kb, slot_b)
            compute(kb, slot_b)
            fire_scatter(slot_b)
            drain_scatter(slot_a)

            @pl.when(k + 1 < NPAIR)
            def _():
                fire_gather(ka + 2, slot_a)

            drain_scatter(slot_b)
            return carry

        lax.fori_loop(0, NPAIR, pair, 0)
        plsc.subcore_barrier()

        for r in range(RPT // W):
            rows = pl.ds(s * RPT + r * W, W)
            pltpu.sync_copy(acc_s.at[rows], rowbuf_a)
            pltpu.sync_copy(rowbuf_a, num_out.at[p, c, rows])

        if p == 0:
            zero_rowbuf(rowbuf_a)
            zero_acc_rows()
            plsc.subcore_barrier()

    for r in range(RPT // W):
        rows = pl.ds(s * RPT + r * W, W)
        pltpu.sync_copy(den_s.at[rows], dbuf_a)
        pltpu.sync_copy(dbuf_a, den_out.at[c, rows])


@functools.cache
def _make_sc_edges():
    return pl.kernel(
        _sc_body,
        mesh=plsc.VectorSubcoreMesh(core_axis_name="c", subcore_axis_name="s"),
        compiler_params=pltpu.CompilerParams(
            needs_layout_passes=False, use_tc_tiling_on_sc=False),
        out_type=[
            jax.ShapeDtypeStruct((2, 2, NP, 128), jnp.float32),
            jax.ShapeDtypeStruct((2, NP, 16), jnp.float32),
        ],
        scratch_types=[
            pltpu.VMEM((NWIN, 1, W), jnp.int32),
            pltpu.VMEM((NWIN, 1, W), jnp.int32),
            pltpu.VMEM((W, 128), jnp.float32),
            pltpu.VMEM((W, 128), jnp.float32),
            pltpu.VMEM((W, 16), jnp.float32),
            pltpu.VMEM((W, 16), jnp.float32),
            pltpu.VMEM((W, 16), jnp.float32),
            pltpu.VMEM((W, 16), jnp.float32),
            pltpu.VMEM((W,), jnp.int32),
            pltpu.VMEM((W,), jnp.int32),
            pltpu.VMEM((W, 16), jnp.float32),
            pltpu.VMEM((W, 16), jnp.float32),
            pltpu.VMEM((2 * W,), jnp.float32),
            pltpu.VMEM_SHARED((NP, 128), jnp.float32),
            pltpu.VMEM_SHARED((NP, 16), jnp.float32),
            pltpu.SemaphoreType.DMA,
            pltpu.SemaphoreType.DMA,
            pltpu.SemaphoreType.DMA,
            pltpu.SemaphoreType.DMA,
        ],
    )


def _finish_body(num_ref, den_ref, bias_ref, out_ref):
    bm = out_ref.shape[0]
    n0 = num_ref[0] + num_ref[1]
    n1 = num_ref[2] + num_ref[3]
    den = den_ref[0] + den_ref[1]
    d0 = jnp.concatenate(
        [jnp.broadcast_to(den[:, h:h + 1], (bm, OUT)) for h in (0, 1)], axis=1)
    d1 = jnp.concatenate(
        [jnp.broadcast_to(den[:, h:h + 1], (bm, OUT)) for h in (2, 3)], axis=1)
    out_ref[...] = jnp.concatenate(
        [n0 / (d0 + 1e-16), n1 / (d1 + 1e-16)], axis=1) + bias_ref[...]


def _finish(num, den, bias2d):
    bm = 400
    grid = N // bm
    return pl.pallas_call(
        _finish_body,
        grid=(grid,),
        in_specs=[
            pl.BlockSpec((4, bm, 128), lambda i: (0, i, 0)),
            pl.BlockSpec((2, bm, 16), lambda i: (0, i, 0)),
            pl.BlockSpec((1, HO), lambda i: (0, 0)),
        ],
        out_specs=pl.BlockSpec((bm, HO), lambda i: (i, 0)),
        out_shape=jax.ShapeDtypeStruct((N, HO), jnp.float32),
    )(num, den, bias2d)


def kernel(x, edge_index, weight, att, bias):
    src0, dst0 = edge_index[0], edge_index[1]
    keep = src0 != dst0
    npv = jnp.asarray(N, src0.dtype)
    src0 = jnp.where(keep, src0, npv)
    dst0 = jnp.where(keep, dst0, npv)
    loop = jnp.arange(N, dtype=src0.dtype)
    npad = EP - (E + N)
    padr = (N + 1 + jnp.arange(npad, dtype=src0.dtype) % (NP - N - 1))
    src = jnp.concatenate([src0, loop, padr])
    dst = jnp.concatenate([dst0, loop, padr])

    att_i = att[0, :, :OUT]
    att_j = att[0, :, OUT:]
    eye = jnp.eye(H, dtype=jnp.float32)
    a_j = (att_j[:, :, None] * eye[:, None, :]).reshape(HO, H)
    a_i = (att_i[:, :, None] * eye[:, None, :]).reshape(HO, H)
    amat = jnp.concatenate(
        [a_j, a_i, jnp.zeros((HO, 8), jnp.float32)], axis=1)

    xpad = jnp.pad(x, ((0, NP - N), (0, 0)))

    xw0, xw1, stab = _dense(xpad, weight, amat)

    src3 = src.astype(jnp.int32).reshape(NW, NWIN, 1, W)
    dst3 = dst.astype(jnp.int32).reshape(NW, NWIN, 1, W)
    num, den = _make_sc_edges()(src3, dst3, stab, xw0, xw1)

    out = _finish(num.reshape(4, NP, 128), den, bias.reshape(1, HO))

    ixz = jnp.zeros((N,), dtype=x.dtype)
    structure_kl_loss = jnp.zeros((), dtype=x.dtype)
    return (out, ixz, structure_kl_loss)
